# baseline (speedup 1.0000x reference)
import functools

import jax
import jax.numpy as jnp
from jax import lax
from jax.experimental import pallas as pl
from jax.experimental.pallas import tpu as pltpu
from jax.experimental.pallas import tpu_sc as plsc

DIM = 2048
HIDDEN = 1024
N_EXP = 16
TOPK = 2
BLK = 128
T = 4096
P = T * TOPK
PADROWS = P + N_EXP * BLK
G_E = PADROWS // BLK
TB_R = 512
TB_S = 256
TB_C = 512

SC_CORES = 2
SC_SUBCORES = 16
SC_WORKERS = SC_CORES * SC_SUBCORES
GATHER_CHUNK = 16


def _router_body(x_ref, gw_ref, w_ref, e_ref):
    xb = x_ref[...]
    logits = lax.dot_general(xb, gw_ref[...], (((1,), (1,)), ((), ())),
                             preferred_element_type=jnp.float32)
    iota = lax.broadcasted_iota(jnp.int32, logits.shape, 1)
    m1 = jnp.max(logits, axis=1, keepdims=True)
    e1 = jnp.min(jnp.where(logits == m1, iota, N_EXP), axis=1, keepdims=True)
    masked = jnp.where(iota == e1, -jnp.inf, logits)
    m2 = jnp.max(masked, axis=1, keepdims=True)
    e2 = jnp.min(jnp.where(masked == m2, iota, N_EXP), axis=1, keepdims=True)
    a2 = jnp.exp(m2 - m1)
    s = 1.0 + a2
    w_ref[:, 0:1] = 1.0 / s
    w_ref[:, 1:2] = a2 / s
    e_ref[:, 0:1] = e1
    e_ref[:, 1:2] = e2


def _router(x_flat, gate_w):
    return pl.pallas_call(
        _router_body,
        grid=(T // TB_R,),
        in_specs=[
            pl.BlockSpec((TB_R, DIM), lambda i: (i, 0)),
            pl.BlockSpec((N_EXP, DIM), lambda i: (0, 0)),
        ],
        out_specs=[
            pl.BlockSpec((TB_R, 128), lambda i: (i, 0)),
            pl.BlockSpec((TB_R, 128), lambda i: (i, 0)),
        ],
        out_shape=[
            jax.ShapeDtypeStruct((T, 128), jnp.float32),
            jax.ShapeDtypeStruct((T, 128), jnp.int32),
        ],
    )(x_flat, gate_w)


def _sc_gather(table, idx, n_rows, dim):
    rpw = n_rows // SC_WORKERS
    nchunk = rpw // GATHER_CHUNK
    mesh = plsc.VectorSubcoreMesh(core_axis_name="c", subcore_axis_name="s")

    @functools.partial(
        pl.kernel,
        out_type=jax.ShapeDtypeStruct((n_rows, dim), jnp.float32),
        mesh=mesh,
        scratch_types=[
            pltpu.VMEM((rpw,), jnp.int32),
            pltpu.VMEM((GATHER_CHUNK, dim), jnp.float32),
            pltpu.SemaphoreType.DMA,
        ],
    )
    def k(table_hbm, idx_hbm, out_hbm, idx_v, rows_v, sem):
        wid = lax.axis_index("s") * SC_CORES + lax.axis_index("c")
        base = wid * rpw
        pltpu.sync_copy(idx_hbm.at[pl.ds(base, rpw)], idx_v)

        def chunk(c, carry):
            off = c * GATHER_CHUNK
            pltpu.async_copy(
                table_hbm.at[idx_v.at[pl.ds(off, GATHER_CHUNK)]], rows_v, sem
            ).wait()
            pltpu.sync_copy(rows_v, out_hbm.at[pl.ds(base + off, GATHER_CHUNK)])
            return carry

        lax.fori_loop(0, nchunk, chunk, 0)

    return k(table, idx)


def _grouped_body(be_ref, vl_ref, xg_ref, w1_ref, w3_ref, w2_ref, out_ref):
    g = pl.program_id(0)

    @pl.when(vl_ref[g] == 1)
    def _():
        xb = xg_ref[...].astype(jnp.bfloat16)
        a = lax.dot_general(xb, w1_ref[0].astype(jnp.bfloat16),
                            (((1,), (1,)), ((), ())),
                            preferred_element_type=jnp.float32)
        b = lax.dot_general(xb, w3_ref[0].astype(jnp.bfloat16),
                            (((1,), (1,)), ((), ())),
                            preferred_element_type=jnp.float32)
        h = (jax.nn.silu(a) * b).astype(jnp.bfloat16)
        out_ref[...] = lax.dot_general(h, w2_ref[0].astype(jnp.bfloat16),
                                       (((1,), (1,)), ((), ())),
                                       preferred_element_type=jnp.float32)

    @pl.when(vl_ref[g] == 0)
    def _():
        out_ref[...] = jnp.zeros_like(out_ref)


def _grouped_gemm(block_expert, valid, xg, w1, w3, w2):
    grid_spec = pltpu.PrefetchScalarGridSpec(
        num_scalar_prefetch=2,
        grid=(G_E,),
        in_specs=[
            pl.BlockSpec((BLK, DIM), lambda g, be, vl: (g, 0)),
            pl.BlockSpec((1, HIDDEN, DIM), lambda g, be, vl: (be[g], 0, 0)),
            pl.BlockSpec((1, HIDDEN, DIM), lambda g, be, vl: (be[g], 0, 0)),
            pl.BlockSpec((1, DIM, HIDDEN), lambda g, be, vl: (be[g], 0, 0)),
        ],
        out_specs=pl.BlockSpec((BLK, DIM), lambda g, be, vl: (g, 0)),
    )
    return pl.pallas_call(
        _grouped_body,
        grid_spec=grid_spec,
        out_shape=jax.ShapeDtypeStruct((PADROWS, DIM), jnp.float32),
    )(block_expert, valid, xg, w1, w3, w2)


def _shared_body(x_ref, sw1_ref, sw3_ref, sw2_ref, out_ref):
    xb = x_ref[...].astype(jnp.bfloat16)
    a = lax.dot_general(xb, sw1_ref[...].astype(jnp.bfloat16),
                        (((1,), (1,)), ((), ())),
                        preferred_element_type=jnp.float32)
    b = lax.dot_general(xb, sw3_ref[...].astype(jnp.bfloat16),
                        (((1,), (1,)), ((), ())),
                        preferred_element_type=jnp.float32)
    h = (jax.nn.silu(a) * b).astype(jnp.bfloat16)
    out_ref[...] = lax.dot_general(h, sw2_ref[...].astype(jnp.bfloat16),
                                   (((1,), (1,)), ((), ())),
                                   preferred_element_type=jnp.float32)


def _shared_expert(x_flat, sw1, sw3, sw2):
    return pl.pallas_call(
        _shared_body,
        grid=(T // TB_S,),
        in_specs=[
            pl.BlockSpec((TB_S, DIM), lambda i: (i, 0)),
            pl.BlockSpec((HIDDEN, DIM), lambda i: (0, 0)),
            pl.BlockSpec((HIDDEN, DIM), lambda i: (0, 0)),
            pl.BlockSpec((DIM, HIDDEN), lambda i: (0, 0)),
        ],
        out_specs=pl.BlockSpec((TB_S, DIM), lambda i: (i, 0)),
        out_shape=jax.ShapeDtypeStruct((T, DIM), jnp.float32),
    )(x_flat, sw1, sw3, sw2)


def _combine_body(y_ref, w_ref, sh_ref, out_ref):
    y0 = y_ref[:, 0, :]
    y1 = y_ref[:, 1, :]
    w0 = w_ref[:, 0:1]
    w1 = w_ref[:, 1:2]
    out_ref[...] = y0 * w0 + y1 * w1 + sh_ref[...]


def _combine(y2, w_out, shared):
    return pl.pallas_call(
        _combine_body,
        grid=(T // TB_C,),
        in_specs=[
            pl.BlockSpec((TB_C, TOPK, DIM), lambda i: (i, 0, 0)),
            pl.BlockSpec((TB_C, 128), lambda i: (i, 0)),
            pl.BlockSpec((TB_C, DIM), lambda i: (i, 0)),
        ],
        out_specs=pl.BlockSpec((TB_C, DIM), lambda i: (i, 0)),
        out_shape=jax.ShapeDtypeStruct((T, DIM), jnp.float32),
    )(y2, w_out, shared)


def kernel(x, gate_w, w1, w3, w2, sw1, sw3, sw2):
    bsz, seq, dim = x.shape
    x_flat = x.reshape(-1, dim)

    w_out, e_out = _router(x_flat, gate_w)

    e_p = e_out[:, :TOPK].reshape(-1)
    onehot = (e_p[:, None] == jnp.arange(N_EXP)[None, :]).astype(jnp.int32)
    cum = jnp.cumsum(onehot, axis=0)
    rank = jnp.sum(cum * onehot, axis=1) - 1
    counts = cum[-1]
    padded = ((counts + BLK - 1) // BLK) * BLK
    pad_start = jnp.concatenate(
        [jnp.zeros((1,), jnp.int32), jnp.cumsum(padded)[:-1].astype(jnp.int32)])
    row_p = pad_start[e_p] + rank
    src_tok = jnp.zeros((PADROWS,), jnp.int32).at[row_p].set(
        jnp.arange(P, dtype=jnp.int32) // TOPK)
    nblocks = padded // BLK
    blk_cum = jnp.cumsum(nblocks)
    gids = jnp.arange(G_E, dtype=jnp.int32)
    be = jnp.searchsorted(blk_cum, gids, side="right").astype(jnp.int32)
    valid = (gids < blk_cum[-1]).astype(jnp.int32)
    block_expert = jnp.minimum(be, N_EXP - 1)

    xg = _sc_gather(x_flat, src_tok, PADROWS, dim)

    go = _grouped_gemm(block_expert, valid, xg, w1, w3, w2)
    shared = _shared_expert(x_flat, sw1, sw3, sw2)

    y2 = _sc_gather(go, row_p, P, dim).reshape(T, TOPK, dim)

    out = _combine(y2, w_out, shared)
    return out.reshape(bsz, seq, dim)


# speedup vs baseline: 1.1358x; 1.1358x over previous
import functools

import jax
import jax.numpy as jnp
from jax import lax
from jax.experimental import pallas as pl
from jax.experimental.pallas import tpu as pltpu
from jax.experimental.pallas import tpu_sc as plsc

DIM = 2048
HIDDEN = 1024
N_EXP = 16
TOPK = 2
BLK = 128
T = 4096
P = T * TOPK
PADROWS = P + N_EXP * BLK
G_E = PADROWS // BLK
TB_R = 512
TB_S = 256
TB_C = 512

SC_CORES = 2
SC_SUBCORES = 16
SC_WORKERS = SC_CORES * SC_SUBCORES
GATHER_CHUNK = 16


def _router_body(x_ref, gw_ref, w_ref, e_ref):
    xb = x_ref[...]
    logits = lax.dot_general(xb, gw_ref[...], (((1,), (1,)), ((), ())),
                             preferred_element_type=jnp.float32)
    iota = lax.broadcasted_iota(jnp.int32, logits.shape, 1)
    m1 = jnp.max(logits, axis=1, keepdims=True)
    e1 = jnp.min(jnp.where(logits == m1, iota, N_EXP), axis=1, keepdims=True)
    masked = jnp.where(iota == e1, -jnp.inf, logits)
    m2 = jnp.max(masked, axis=1, keepdims=True)
    e2 = jnp.min(jnp.where(masked == m2, iota, N_EXP), axis=1, keepdims=True)
    a2 = jnp.exp(m2 - m1)
    s = 1.0 + a2
    w_ref[:, 0:1] = 1.0 / s
    w_ref[:, 1:2] = a2 / s
    e_ref[:, 0:1] = e1
    e_ref[:, 1:2] = e2


def _router(x_flat, gate_w):
    return pl.pallas_call(
        _router_body,
        grid=(T // TB_R,),
        in_specs=[
            pl.BlockSpec((TB_R, DIM), lambda i: (i, 0)),
            pl.BlockSpec((N_EXP, DIM), lambda i: (0, 0)),
        ],
        out_specs=[
            pl.BlockSpec((TB_R, 128), lambda i: (i, 0)),
            pl.BlockSpec((TB_R, 128), lambda i: (i, 0)),
        ],
        out_shape=[
            jax.ShapeDtypeStruct((T, 128), jnp.float32),
            jax.ShapeDtypeStruct((T, 128), jnp.int32),
        ],
    )(x_flat, gate_w)


def _sc_gather(table, idx, n_rows, dim):
    rpw = n_rows // SC_WORKERS
    nchunk = rpw // GATHER_CHUNK
    mesh = plsc.VectorSubcoreMesh(core_axis_name="c", subcore_axis_name="s")

    @functools.partial(
        pl.kernel,
        out_type=jax.ShapeDtypeStruct((n_rows, dim), jnp.float32),
        mesh=mesh,
        scratch_types=[
            pltpu.VMEM((rpw,), jnp.int32),
            pltpu.VMEM((GATHER_CHUNK, dim), jnp.float32),
            pltpu.SemaphoreType.DMA,
        ],
    )
    def k(table_hbm, idx_hbm, out_hbm, idx_v, rows_v, sem):
        wid = lax.axis_index("s") * SC_CORES + lax.axis_index("c")
        base = wid * rpw
        pltpu.sync_copy(idx_hbm.at[pl.ds(base, rpw)], idx_v)

        def chunk(c, carry):
            off = c * GATHER_CHUNK
            pltpu.async_copy(
                table_hbm.at[idx_v.at[pl.ds(off, GATHER_CHUNK)]], rows_v, sem
            ).wait()
            pltpu.sync_copy(rows_v, out_hbm.at[pl.ds(base + off, GATHER_CHUNK)])
            return carry

        lax.fori_loop(0, nchunk, chunk, 0)

    return k(table, idx)


def _grouped_body(be_ref, vl_ref, xg_ref, w1_ref, w3_ref, w2_ref, out_ref):
    g = pl.program_id(0)

    @pl.when(vl_ref[g] == 1)
    def _():
        xb = xg_ref[...].astype(jnp.bfloat16)
        a = lax.dot_general(xb, w1_ref[0].astype(jnp.bfloat16),
                            (((1,), (1,)), ((), ())),
                            preferred_element_type=jnp.float32)
        b = lax.dot_general(xb, w3_ref[0].astype(jnp.bfloat16),
                            (((1,), (1,)), ((), ())),
                            preferred_element_type=jnp.float32)
        h = (jax.nn.silu(a) * b).astype(jnp.bfloat16)
        out_ref[...] = lax.dot_general(h, w2_ref[0].astype(jnp.bfloat16),
                                       (((1,), (1,)), ((), ())),
                                       preferred_element_type=jnp.float32)

    @pl.when(vl_ref[g] == 0)
    def _():
        out_ref[...] = jnp.zeros_like(out_ref)


def _grouped_gemm(block_expert, valid, xg, w1, w3, w2):
    grid_spec = pltpu.PrefetchScalarGridSpec(
        num_scalar_prefetch=2,
        grid=(G_E,),
        in_specs=[
            pl.BlockSpec((BLK, DIM), lambda g, be, vl: (g, 0)),
            pl.BlockSpec((1, HIDDEN, DIM), lambda g, be, vl: (be[g], 0, 0)),
            pl.BlockSpec((1, HIDDEN, DIM), lambda g, be, vl: (be[g], 0, 0)),
            pl.BlockSpec((1, DIM, HIDDEN), lambda g, be, vl: (be[g], 0, 0)),
        ],
        out_specs=pl.BlockSpec((BLK, DIM), lambda g, be, vl: (g, 0)),
    )
    return pl.pallas_call(
        _grouped_body,
        grid_spec=grid_spec,
        out_shape=jax.ShapeDtypeStruct((PADROWS, DIM), jnp.float32),
    )(block_expert, valid, xg, w1, w3, w2)


def _shared_body(x_ref, sw1_ref, sw3_ref, sw2_ref, out_ref):
    xb = x_ref[...].astype(jnp.bfloat16)
    a = lax.dot_general(xb, sw1_ref[...].astype(jnp.bfloat16),
                        (((1,), (1,)), ((), ())),
                        preferred_element_type=jnp.float32)
    b = lax.dot_general(xb, sw3_ref[...].astype(jnp.bfloat16),
                        (((1,), (1,)), ((), ())),
                        preferred_element_type=jnp.float32)
    h = (jax.nn.silu(a) * b).astype(jnp.bfloat16)
    out_ref[...] = lax.dot_general(h, sw2_ref[...].astype(jnp.bfloat16),
                                   (((1,), (1,)), ((), ())),
                                   preferred_element_type=jnp.float32)


def _shared_expert(x_flat, sw1, sw3, sw2):
    return pl.pallas_call(
        _shared_body,
        grid=(T // TB_S,),
        in_specs=[
            pl.BlockSpec((TB_S, DIM), lambda i: (i, 0)),
            pl.BlockSpec((HIDDEN, DIM), lambda i: (0, 0)),
            pl.BlockSpec((HIDDEN, DIM), lambda i: (0, 0)),
            pl.BlockSpec((DIM, HIDDEN), lambda i: (0, 0)),
        ],
        out_specs=pl.BlockSpec((TB_S, DIM), lambda i: (i, 0)),
        out_shape=jax.ShapeDtypeStruct((T, DIM), jnp.float32),
    )(x_flat, sw1, sw3, sw2)


def _combine_body(y0_ref, y1_ref, w_ref, sh_ref, out_ref):
    w0 = w_ref[:, 0:1]
    w1 = w_ref[:, 1:2]
    out_ref[...] = y0_ref[...] * w0 + y1_ref[...] * w1 + sh_ref[...]


def _combine(y2, w_out, shared):
    nb = T // TB_C
    return pl.pallas_call(
        _combine_body,
        grid=(nb,),
        in_specs=[
            pl.BlockSpec((TB_C, DIM), lambda i: (i, 0)),
            pl.BlockSpec((TB_C, DIM), lambda i, _nb=nb: (i + _nb, 0)),
            pl.BlockSpec((TB_C, 128), lambda i: (i, 0)),
            pl.BlockSpec((TB_C, DIM), lambda i: (i, 0)),
        ],
        out_specs=pl.BlockSpec((TB_C, DIM), lambda i: (i, 0)),
        out_shape=jax.ShapeDtypeStruct((T, DIM), jnp.float32),
    )(y2, y2, w_out, shared)


def kernel(x, gate_w, w1, w3, w2, sw1, sw3, sw2):
    bsz, seq, dim = x.shape
    x_flat = x.reshape(-1, dim)

    w_out, e_out = _router(x_flat, gate_w)

    e_p = e_out[:, :TOPK].reshape(-1)
    oh = (e_p[:, None] == jnp.arange(N_EXP)[None, :]).astype(jnp.float32)
    ohb = oh.reshape(P // BLK, BLK, N_EXP)
    bs = ohb.sum(axis=1)
    excl_blk = jnp.cumsum(bs, axis=0) - bs
    tri = jnp.tril(jnp.ones((BLK, BLK), jnp.float32))
    within = jax.lax.dot_general(
        tri, ohb, (((1,), (1,)), ((), ())))
    cum_incl = within.transpose(1, 0, 2) + excl_blk[:, None, :]
    rank = (cum_incl.reshape(P, N_EXP) * oh).sum(axis=1) - 1.0
    counts = bs.sum(axis=0)
    padded = jnp.ceil(counts / BLK) * BLK
    pad_start = jnp.cumsum(padded) - padded
    row_p = (oh @ pad_start + rank).astype(jnp.int32)
    src_tok = jnp.zeros((PADROWS,), jnp.int32).at[row_p].set(
        jnp.arange(P, dtype=jnp.int32) // TOPK)
    blk_cum = jnp.cumsum(padded) * (1.0 / BLK)
    gids = jnp.arange(G_E, dtype=jnp.float32)
    be = (gids[:, None] >= blk_cum[None, :]).sum(axis=1).astype(jnp.int32)
    valid = (gids < blk_cum[-1]).astype(jnp.int32)
    block_expert = jnp.minimum(be, N_EXP - 1)

    xg = _sc_gather(x_flat, src_tok, PADROWS, dim)

    go = _grouped_gemm(block_expert, valid, xg, w1, w3, w2)
    shared = _shared_expert(x_flat, sw1, sw3, sw2)

    idx2 = jnp.concatenate([row_p[0::TOPK], row_p[1::TOPK]])
    y2 = _sc_gather(go, idx2, P, dim)

    out = _combine(y2, w_out, shared)
    return out.reshape(bsz, seq, dim)
